# baseline (speedup 1.0000x reference)
import functools
import math

import jax
import jax.numpy as jnp
from jax import lax
from jax.experimental import pallas as pl
from jax.experimental.pallas import tpu as pltpu
from jax.experimental.pallas import tpu_sc as plsc

_SPARSITY = 0.05
_L = 16
_NC = 2
_NS = 16
_NW = _NC * _NS

_IMIN = -2147483648


def _skey(v):
    b = lax.bitcast_convert_type(v, jnp.int32)
    return b ^ lax.shift_right_logical(lax.shift_right_arithmetic(b, 31), 1)


def _scan_hist(hist, nchunks, base, targets, iota, zeros):
    def body(i, carry):
        t, accs = carry
        j = nchunks - 1 - i
        sl = pl.ds(base + j * _L, _L)
        h = hist[sl]
        hist[sl] = zeros
        rc = lax.rev(plsc.cumsum(lax.rev(h, (0,))), (0,))
        g = t + rc
        ga = g - h
        idx = j * _L + iota
        new_accs = []
        for (b_acc, ca_acc, nb_acc, f_acc), r in zip(accs, targets):
            ci = ((g >= r) & (ga < r)).astype(jnp.int32)
            new_accs.append((
                b_acc + ci * idx,
                ca_acc + ci * ga,
                nb_acc + ci * h,
                f_acc + ci,
            ))
        return g[0], tuple(new_accs)

    zero_acc = (iota * 0,) * 4
    t, accs = lax.fori_loop(
        0, nchunks, body, (jnp.int32(0), (zero_acc,) * len(targets))
    )
    out = []
    for b_acc, ca_acc, nb_acc, f_acc in accs:
        out.append(
            (jnp.sum(f_acc), jnp.sum(b_acc), jnp.sum(ca_acc), jnp.sum(nb_acc))
        )
    return out


def _sc_body(x_hbm, o_hbm, xbuf, cand, hist, *, n, k, rows_per_worker):
    wid = lax.axis_index("s") * _NC + lax.axis_index("c")
    iota = lax.iota(jnp.int32, _L)
    zeros = iota * 0
    ones = zeros + 1
    nchunks = n // _L

    def zbody(i, _):
        hist[pl.ds(i * _L, _L)] = zeros
        return 0

    lax.fori_loop(0, hist.shape[0] // _L, zbody, 0)

    for rr in range(rows_per_worker):
        row = wid * rows_per_worker + rr
        pltpu.sync_copy(x_hbm.at[row], xbuf)

        def p1(i, _):
            for u in range(8):
                v = xbuf[pl.ds(i * 128 + u * _L, _L)]
                sk = _skey(v)
                ub = lax.shift_right_logical(sk, 20) ^ 0x800
                plsc.addupdate_scatter(hist, [ub], ones)
            return 0

        lax.fori_loop(0, nchunks // 8, p1, 0)

        ((_, b1, ca1, n1),) = _scan_hist(
            hist, 4096 // _L, 0, (jnp.int32(k),), iota, zeros
        )

        def p2(i, carry):
            off, mb = carry
            for u in range(8):
                v = xbuf[pl.ds(i * 128 + u * _L, _L)]
                sk = _skey(v)
                ub = lax.shift_right_logical(sk, 20) ^ 0x800
                mb = jnp.maximum(mb, jnp.where(ub < b1, sk, _IMIN))
                match = ub == b1
                plsc.store_compressed(cand.at[pl.ds(off, _L)], sk, mask=match)
                off = off + plsc.all_reduce_population_count(match)[0]
            return off, mb

        _, mb1 = lax.fori_loop(0, nchunks // 8, p2, (jnp.int32(0), zeros + _IMIN))

        nch1 = lax.div(n1 + (_L - 1), jnp.int32(_L))

        def p3(i, _):
            sk = cand[pl.ds(i * _L, _L)]
            valid = (i * _L + iota) < n1
            ub2 = lax.shift_right_logical(sk, 8) & 0xFFF
            plsc.addupdate_scatter(hist, [ub2], ones, mask=valid)
            return 0

        lax.fori_loop(0, nch1, p3, 0)

        r1 = k - ca1
        ((_, b2, ca2, n2),) = _scan_hist(hist, 4096 // _L, 0, (r1,), iota, zeros)

        def p4(i, mb):
            sk = cand[pl.ds(i * _L, _L)]
            valid = (i * _L + iota) < n1
            ub2 = lax.shift_right_logical(sk, 8) & 0xFFF
            mb = jnp.maximum(mb, jnp.where(valid & (ub2 < b2), sk, _IMIN))
            ub3 = sk & 0xFF
            plsc.addupdate_scatter(hist, [ub3], ones, mask=valid & (ub2 == b2))
            return mb

        mb2 = lax.fori_loop(0, nch1, p4, zeros + _IMIN)

        r2 = r1 - ca2
        (_, lo_k, _, _), (fnd_k1, lo_k1, _, _) = _scan_hist(
            hist, 256 // _L, 0, (r2, r2 + 1), iota, zeros
        )

        prefix = ((b1 ^ 0x800) << 20) | (b2 << 8)
        sk_k = prefix | lo_k
        mb = jnp.maximum(jnp.max(mb1), jnp.max(mb2))
        sk_k1 = jnp.where(fnd_k1 > 0, prefix | lo_k1, mb)

        def unkey(s):
            b = s ^ lax.shift_right_logical(lax.shift_right_arithmetic(s, 31), 1)
            return lax.bitcast_convert_type(b, jnp.float32)

        thr = (unkey(sk_k) + unkey(sk_k1)) * jnp.float32(0.5)

        def p5(i, _):
            for u in range(8):
                sl = pl.ds(i * 128 + u * _L, _L)
                v = xbuf[sl]
                xbuf[sl] = jnp.where(v > thr, jnp.float32(1.0), jnp.float32(0.0))
            return 0

        lax.fori_loop(0, nchunks // 8, p5, 0)

        pltpu.sync_copy(xbuf, o_hbm.at[row])


def kernel(x):
    m, n = x.shape
    k = math.ceil(_SPARSITY * n)
    rows_per_worker = m // _NW
    mesh = plsc.VectorSubcoreMesh(
        core_axis_name="c", subcore_axis_name="s", num_cores=_NC, num_subcores=_NS
    )
    body = functools.partial(_sc_body, n=n, k=k, rows_per_worker=rows_per_worker)
    return pl.kernel(
        body,
        out_type=jax.ShapeDtypeStruct((m, n), jnp.float32),
        mesh=mesh,
        compiler_params=pltpu.CompilerParams(needs_layout_passes=False),
        scratch_types=[
            pltpu.VMEM((n,), jnp.float32),
            pltpu.VMEM((n + _L,), jnp.int32),
            pltpu.VMEM((4096,), jnp.int32),
        ],
    )(x)


# speedup vs baseline: 1.0207x; 1.0207x over previous
import functools
import math

import jax
import jax.numpy as jnp
from jax import lax
from jax.experimental import pallas as pl
from jax.experimental.pallas import tpu as pltpu
from jax.experimental.pallas import tpu_sc as plsc

_SPARSITY = 0.05
_L = 16
_NC = 2
_NS = 16
_NW = _NC * _NS

_IMIN = -2147483648


def _suffix(v):
    return lax.rev(plsc.cumsum(lax.rev(v, (0,))), (0,))


def _cross(g, ga, r):
    return ((g >= r) & (ga < r)).astype(jnp.int32)


def _scan4096(hist, r, iota, zeros):
    seg_stride = iota * 256

    def l0(j, seg):
        return seg + plsc.load_gather(hist, [seg_stride + j])

    seg = lax.fori_loop(0, 256, l0, zeros)
    rc = _suffix(seg)
    ga = rc - seg
    ci = _cross(rc, ga, r)
    s1 = jnp.sum(ci * iota)
    t1 = jnp.sum(ci * ga)

    base1 = s1 * 256

    def l1(m, sub):
        return sub + plsc.load_gather(hist, [base1 + iota * _L + m])

    sub = lax.fori_loop(0, _L, l1, zeros)
    rc2 = t1 + _suffix(sub)
    ga2 = rc2 - sub
    ci2 = _cross(rc2, ga2, r)
    s2 = jnp.sum(ci2 * iota)
    t2 = jnp.sum(ci2 * ga2)

    base2 = base1 + s2 * _L
    h = plsc.load_gather(hist, [base2 + iota])
    rc3 = t2 + _suffix(h)
    ga3 = rc3 - h
    ci3 = _cross(rc3, ga3, r)
    b = jnp.sum(ci3 * (base2 + iota))
    return b, jnp.sum(ci3 * ga3), jnp.sum(ci3 * h)


def _scan256(hist, r, iota, zeros):
    seg_stride = iota * _L

    def l0(j, seg):
        return seg + plsc.load_gather(hist, [seg_stride + j])

    seg = lax.fori_loop(0, _L, l0, zeros)
    rc = _suffix(seg)
    ga = rc - seg
    ci = _cross(rc, ga, r)
    s1 = jnp.sum(ci * iota)
    t1 = jnp.sum(ci * ga)

    h = plsc.load_gather(hist, [s1 * _L + iota])
    rc2 = t1 + _suffix(h)
    ga2 = rc2 - h
    ci2 = _cross(rc2, ga2, r)
    b = jnp.sum(ci2 * (s1 * _L + iota))
    return jnp.sum(ci2), b, jnp.sum(ci2 * ga2)


def _sc_body(x_hbm, o_hbm, xbuf, skeybuf, cand, hist, *, n, k, rows_per_worker):
    wid = lax.axis_index("s") * _NC + lax.axis_index("c")
    iota = lax.iota(jnp.int32, _L)
    zeros = iota * 0
    ones = zeros + 1
    seg_cap = n // _L
    lanebase = iota * seg_cap
    nchunks = n // _L

    def zero_hist(nbkt):
        def z(i, _):
            hist[pl.ds(i * _L, _L)] = zeros
            return 0

        lax.fori_loop(0, nbkt // _L, z, 0)

    zero_hist(4096)

    for rr in range(rows_per_worker):
        row = wid * rows_per_worker + rr
        pltpu.sync_copy(x_hbm.at[row], xbuf)

        def p1(i, _):
            for u in range(8):
                sl = pl.ds(i * 128 + u * _L, _L)
                v = xbuf[sl]
                b = lax.bitcast_convert_type(v, jnp.int32)
                sk = b ^ lax.shift_right_logical(
                    lax.shift_right_arithmetic(b, 31), 1
                )
                skeybuf[sl] = sk
                ub = lax.shift_right_logical(sk, 20) ^ 0x800
                plsc.addupdate_scatter(hist, [ub], ones)
            return 0

        lax.fori_loop(0, nchunks // 8, p1, 0)

        b1, ca1, n1 = _scan4096(hist, jnp.int32(k), iota, zeros)
        zero_hist(4096)

        def p2(i, carry):
            off16, mb = carry
            for u in range(8):
                sk = skeybuf[pl.ds(i * 128 + u * _L, _L)]
                ub = lax.shift_right_logical(sk, 20) ^ 0x800
                mb = jnp.maximum(mb, jnp.where(ub < b1, sk, _IMIN))
                match = ub == b1
                plsc.store_scatter(cand, [lanebase + off16], sk, mask=match)
                off16 = off16 + match.astype(jnp.int32)
            return off16, mb

        off16, mb1 = lax.fori_loop(
            0, nchunks // 8, p2, (zeros, zeros + _IMIN)
        )
        len_max = jnp.max(off16)

        def p3(i, _):
            sk = plsc.load_gather(cand, [lanebase + i])
            valid = i < off16
            ub2 = lax.shift_right_logical(sk, 8) & 0xFFF
            plsc.addupdate_scatter(hist, [ub2], ones, mask=valid)
            return 0

        lax.fori_loop(0, len_max, p3, 0)

        r1 = k - ca1
        b2, ca2, n2 = _scan4096(hist, r1, iota, zeros)
        zero_hist(4096)

        def p4(i, mb):
            sk = plsc.load_gather(cand, [lanebase + i])
            valid = i < off16
            ub2 = lax.shift_right_logical(sk, 8) & 0xFFF
            mb = jnp.maximum(mb, jnp.where(valid & (ub2 < b2), sk, _IMIN))
            plsc.addupdate_scatter(hist, [sk & 0xFF], ones, mask=valid & (ub2 == b2))
            return mb

        mb2 = lax.fori_loop(0, len_max, p4, zeros + _IMIN)

        r2 = r1 - ca2
        _, lo_k, _ = _scan256(hist, r2, iota, zeros)
        fnd_k1, lo_k1, _ = _scan256(hist, r2 + 1, iota, zeros)
        zero_hist(256)

        prefix = ((b1 ^ 0x800) << 20) | (b2 << 8)
        sk_k = prefix | lo_k
        mb = jnp.maximum(jnp.max(mb1), jnp.max(mb2))
        sk_k1 = jnp.where(fnd_k1 > 0, prefix | lo_k1, mb)

        def unkey(s):
            b = s ^ lax.shift_right_logical(lax.shift_right_arithmetic(s, 31), 1)
            return lax.bitcast_convert_type(b, jnp.float32)

        thr = (unkey(sk_k) + unkey(sk_k1)) * jnp.float32(0.5)

        def p5(i, _):
            for u in range(8):
                sl = pl.ds(i * 128 + u * _L, _L)
                v = xbuf[sl]
                xbuf[sl] = jnp.where(v > thr, jnp.float32(1.0), jnp.float32(0.0))
            return 0

        lax.fori_loop(0, nchunks // 8, p5, 0)

        pltpu.sync_copy(xbuf, o_hbm.at[row])


def kernel(x):
    m, n = x.shape
    k = math.ceil(_SPARSITY * n)
    rows_per_worker = m // _NW
    mesh = plsc.VectorSubcoreMesh(
        core_axis_name="c", subcore_axis_name="s", num_cores=_NC, num_subcores=_NS
    )
    body = functools.partial(_sc_body, n=n, k=k, rows_per_worker=rows_per_worker)
    return pl.kernel(
        body,
        out_type=jax.ShapeDtypeStruct((m, n), jnp.float32),
        mesh=mesh,
        compiler_params=pltpu.CompilerParams(needs_layout_passes=False),
        scratch_types=[
            pltpu.VMEM((n,), jnp.float32),
            pltpu.VMEM((n,), jnp.int32),
            pltpu.VMEM((n,), jnp.int32),
            pltpu.VMEM((4096,), jnp.int32),
        ],
    )(x)


# speedup vs baseline: 1.0696x; 1.0479x over previous
import functools
import math

import jax
import jax.numpy as jnp
from jax import lax
from jax.experimental import pallas as pl
from jax.experimental.pallas import tpu as pltpu
from jax.experimental.pallas import tpu_sc as plsc

_SPARSITY = 0.05
_L = 16
_NC = 2
_NS = 16
_NW = _NC * _NS

_IMIN = -2147483648
_SIGN = -2147483648


def _suffix(v):
    return lax.rev(plsc.cumsum(lax.rev(v, (0,))), (0,))


def _cross(g, ga, r):
    return ((g >= r) & (ga < r)).astype(jnp.int32)


def _scan4096(hist, r, iota, zeros):
    seg_stride = iota * 256

    def l0(j, seg):
        return seg + plsc.load_gather(hist, [seg_stride + j])

    seg = lax.fori_loop(0, 256, l0, zeros)
    rc = _suffix(seg)
    ga = rc - seg
    ci = _cross(rc, ga, r)
    s1 = jnp.sum(ci * iota)
    t1 = jnp.sum(ci * ga)

    base1 = s1 * 256

    def l1(m, sub):
        return sub + plsc.load_gather(hist, [base1 + iota * _L + m])

    sub = lax.fori_loop(0, _L, l1, zeros)
    rc2 = t1 + _suffix(sub)
    ga2 = rc2 - sub
    ci2 = _cross(rc2, ga2, r)
    s2 = jnp.sum(ci2 * iota)
    t2 = jnp.sum(ci2 * ga2)

    base2 = base1 + s2 * _L
    h = plsc.load_gather(hist, [base2 + iota])
    rc3 = t2 + _suffix(h)
    ga3 = rc3 - h
    ci3 = _cross(rc3, ga3, r)
    b = jnp.sum(ci3 * (base2 + iota))
    return b, jnp.sum(ci3 * ga3), jnp.sum(ci3 * h)


def _scan256(hist, r, iota, zeros):
    seg_stride = iota * _L

    def l0(j, seg):
        return seg + plsc.load_gather(hist, [seg_stride + j])

    seg = lax.fori_loop(0, _L, l0, zeros)
    rc = _suffix(seg)
    ga = rc - seg
    ci = _cross(rc, ga, r)
    s1 = jnp.sum(ci * iota)
    t1 = jnp.sum(ci * ga)

    h = plsc.load_gather(hist, [s1 * _L + iota])
    rc2 = t1 + _suffix(h)
    ga2 = rc2 - h
    ci2 = _cross(rc2, ga2, r)
    b = jnp.sum(ci2 * (s1 * _L + iota))
    return jnp.sum(ci2), b, jnp.sum(ci2 * ga2)


def _sc_body(x_hbm, o_hbm, xbuf, kbuf, hist, mbs, *, n, k, rows_per_worker):
    wid = lax.axis_index("s") * _NC + lax.axis_index("c")
    iota = lax.iota(jnp.int32, _L)
    zeros = iota * 0
    ones = zeros + 1
    nchunks = n // _L

    def zero_hist(nbkt):
        def z(i, _):
            hist[pl.ds(i * _L, _L)] = zeros
            return 0

        lax.fori_loop(0, nbkt // _L, z, 0)

    zero_hist(4096)

    for rr in range(rows_per_worker):
        row = wid * rows_per_worker + rr
        pltpu.sync_copy(x_hbm.at[row], xbuf)

        def p1(i, _):
            for u in range(8):
                sl = pl.ds(i * 128 + u * _L, _L)
                b = lax.bitcast_convert_type(xbuf[sl], jnp.int32)
                bk = b ^ (lax.shift_right_arithmetic(b, 31) | _SIGN)
                kbuf[sl] = bk
                plsc.addupdate_scatter(
                    hist, [lax.shift_right_logical(bk, 20)], ones
                )
            return 0

        lax.fori_loop(0, nchunks // 8, p1, 0)

        b1, ca1, n1 = _scan4096(hist, jnp.int32(k), iota, zeros)
        zero_hist(4096)

        mbs[pl.ds(0, _L)] = zeros + _IMIN

        @pl.when(ca1 + n1 == k)
        def _():
            def pmb(i, m):
                for u in range(8):
                    bk = kbuf[pl.ds(i * 128 + u * _L, _L)]
                    ub = lax.shift_right_logical(bk, 20)
                    m = jnp.maximum(m, jnp.where(ub < b1, bk ^ _SIGN, _IMIN))
                return m

            mbs[pl.ds(0, _L)] = lax.fori_loop(
                0, nchunks // 8, pmb, zeros + _IMIN
            )

        def p2(i, off):
            for u in range(8):
                bk = kbuf[pl.ds(i * 128 + u * _L, _L)]
                match = lax.shift_right_logical(bk, 20) == b1
                plsc.store_scatter(kbuf, [iota + off * _L], bk, mask=match)
                off = off + match
            return off

        off16 = lax.fori_loop(0, nchunks // 8, p2, zeros)
        len_max = jnp.max(off16)

        def p3(i, _):
            bk = plsc.load_gather(kbuf, [iota + i * _L])
            ub2 = lax.shift_right_logical(bk, 8) & 0xFFF
            plsc.addupdate_scatter(hist, [ub2], ones, mask=i < off16)
            return 0

        lax.fori_loop(0, len_max, p3, 0)

        r1 = k - ca1
        b2, ca2, n2 = _scan4096(hist, r1, iota, zeros)

        def p3z(i, _):
            bk = plsc.load_gather(kbuf, [iota + i * _L])
            ub2 = lax.shift_right_logical(bk, 8) & 0xFFF
            plsc.store_scatter(hist, [ub2], zeros, mask=i < off16)
            return 0

        lax.fori_loop(0, len_max, p3z, 0)

        def p4(i, mb):
            bk = plsc.load_gather(kbuf, [iota + i * _L])
            valid = i < off16
            ub2 = lax.shift_right_logical(bk, 8) & 0xFFF
            mb = jnp.maximum(
                mb, jnp.where(valid & (ub2 < b2), bk ^ _SIGN, _IMIN)
            )
            plsc.addupdate_scatter(
                hist, [bk & 0xFF], ones, mask=valid & (ub2 == b2)
            )
            return mb

        mb2f = lax.fori_loop(0, len_max, p4, zeros + _IMIN)

        r2 = r1 - ca2
        _, lo_k, _ = _scan256(hist, r2, iota, zeros)
        fnd_k1, lo_k1, _ = _scan256(hist, r2 + 1, iota, zeros)

        def p4z(i, _):
            bk = plsc.load_gather(kbuf, [iota + i * _L])
            valid = i < off16
            ub2 = lax.shift_right_logical(bk, 8) & 0xFFF
            plsc.store_scatter(
                hist, [bk & 0xFF], zeros, mask=valid & (ub2 == b2)
            )
            return 0

        lax.fori_loop(0, len_max, p4z, 0)

        prefix = (b1 << 20) | (b2 << 8)
        bk_k = prefix | lo_k
        mbf = jnp.maximum(jnp.max(mbs[pl.ds(0, _L)]), jnp.max(mb2f))
        bk_k1 = jnp.where(fnd_k1 > 0, prefix | lo_k1, mbf ^ _SIGN)

        def unkey(s):
            t = lax.shift_right_arithmetic(s, 31)
            return lax.bitcast_convert_type(s ^ ((t ^ -1) | _SIGN), jnp.float32)

        thr = (unkey(bk_k) + unkey(bk_k1)) * jnp.float32(0.5)

        def p5(i, _):
            for u in range(8):
                sl = pl.ds(i * 128 + u * _L, _L)
                v = xbuf[sl]
                xbuf[sl] = jnp.where(v > thr, jnp.float32(1.0), jnp.float32(0.0))
            return 0

        lax.fori_loop(0, nchunks // 8, p5, 0)

        pltpu.sync_copy(xbuf, o_hbm.at[row])


def kernel(x):
    m, n = x.shape
    k = math.ceil(_SPARSITY * n)
    rows_per_worker = m // _NW
    mesh = plsc.VectorSubcoreMesh(
        core_axis_name="c", subcore_axis_name="s", num_cores=_NC, num_subcores=_NS
    )
    body = functools.partial(_sc_body, n=n, k=k, rows_per_worker=rows_per_worker)
    return pl.kernel(
        body,
        out_type=jax.ShapeDtypeStruct((m, n), jnp.float32),
        mesh=mesh,
        compiler_params=pltpu.CompilerParams(needs_layout_passes=False),
        scratch_types=[
            pltpu.VMEM((n,), jnp.float32),
            pltpu.VMEM((n,), jnp.int32),
            pltpu.VMEM((4096,), jnp.int32),
            pltpu.VMEM((_L,), jnp.int32),
        ],
    )(x)


# speedup vs baseline: 1.6434x; 1.5364x over previous
import functools
import math

import jax
import jax.numpy as jnp
from jax import lax
from jax.experimental import pallas as pl
from jax.experimental.pallas import tpu as pltpu
from jax.experimental.pallas import tpu_sc as plsc

_SPARSITY = 0.05
_L = 16
_NC = 2
_NS = 16
_NW = _NC * _NS

_IMIN = -2147483648
_SIGN = -2147483648

_BLOCK_M = 8


def _suffix(v):
    return lax.rev(plsc.cumsum(lax.rev(v, (0,))), (0,))


def _cross(g, ga, r):
    return ((g >= r) & (ga < r)).astype(jnp.int32)


def _scan4096(hist, r, iota, zeros):
    seg_stride = iota * 256

    def l0(j, seg):
        return seg + plsc.load_gather(hist, [seg_stride + j])

    seg = lax.fori_loop(0, 256, l0, zeros)
    rc = _suffix(seg)
    ga = rc - seg
    ci = _cross(rc, ga, r)
    s1 = jnp.sum(ci * iota)
    t1 = jnp.sum(ci * ga)

    base1 = s1 * 256

    def l1(m, sub):
        return sub + plsc.load_gather(hist, [base1 + iota * _L + m])

    sub = lax.fori_loop(0, _L, l1, zeros)
    rc2 = t1 + _suffix(sub)
    ga2 = rc2 - sub
    ci2 = _cross(rc2, ga2, r)
    s2 = jnp.sum(ci2 * iota)
    t2 = jnp.sum(ci2 * ga2)

    base2 = base1 + s2 * _L
    h = plsc.load_gather(hist, [base2 + iota])
    rc3 = t2 + _suffix(h)
    ga3 = rc3 - h
    ci3 = _cross(rc3, ga3, r)
    b = jnp.sum(ci3 * (base2 + iota))
    return b, jnp.sum(ci3 * ga3), jnp.sum(ci3 * h)


def _scan256(hist, r, iota, zeros):
    seg_stride = iota * _L

    def l0(j, seg):
        return seg + plsc.load_gather(hist, [seg_stride + j])

    seg = lax.fori_loop(0, _L, l0, zeros)
    rc = _suffix(seg)
    ga = rc - seg
    ci = _cross(rc, ga, r)
    s1 = jnp.sum(ci * iota)
    t1 = jnp.sum(ci * ga)

    h = plsc.load_gather(hist, [s1 * _L + iota])
    rc2 = t1 + _suffix(h)
    ga2 = rc2 - h
    ci2 = _cross(rc2, ga2, r)
    b = jnp.sum(ci2 * (s1 * _L + iota))
    return jnp.sum(ci2), b, jnp.sum(ci2 * ga2)


def _sc_body(x_hbm, o_hbm, xbuf, kbuf, hist, mbs, *, n, k, rows_per_worker):
    wid = lax.axis_index("s") * _NC + lax.axis_index("c")
    iota = lax.iota(jnp.int32, _L)
    zeros = iota * 0
    ones = zeros + 1
    nchunks = n // _L

    def zero_hist(nbkt):
        def z(i, _):
            hist[pl.ds(i * _L, _L)] = zeros
            return 0

        lax.fori_loop(0, nbkt // _L, z, 0)

    zero_hist(4096)

    for rr in range(rows_per_worker):
        row = wid * rows_per_worker + rr
        pltpu.sync_copy(x_hbm.at[row], xbuf)

        def p1(i, _):
            for u in range(8):
                sl = pl.ds(i * 128 + u * _L, _L)
                b = lax.bitcast_convert_type(xbuf[sl], jnp.int32)
                bk = b ^ (lax.shift_right_arithmetic(b, 31) | _SIGN)
                kbuf[sl] = bk
                plsc.addupdate_scatter(
                    hist, [lax.shift_right_logical(bk, 20)], ones
                )
            return 0

        lax.fori_loop(0, nchunks // 8, p1, 0)

        b1, ca1, n1 = _scan4096(hist, jnp.int32(k), iota, zeros)
        zero_hist(4096)

        mbs[pl.ds(0, _L)] = zeros + _IMIN

        @pl.when(ca1 + n1 == k)
        def _():
            def pmb(i, m):
                for u in range(8):
                    bk = kbuf[pl.ds(i * 128 + u * _L, _L)]
                    ub = lax.shift_right_logical(bk, 20)
                    m = jnp.maximum(m, jnp.where(ub < b1, bk ^ _SIGN, _IMIN))
                return m

            mbs[pl.ds(0, _L)] = lax.fori_loop(
                0, nchunks // 8, pmb, zeros + _IMIN
            )

        def p2(i, off):
            for u in range(8):
                bk = kbuf[pl.ds(i * 128 + u * _L, _L)]
                match = lax.shift_right_logical(bk, 20) == b1
                plsc.store_scatter(kbuf, [iota + off * _L], bk, mask=match)
                off = off + match
            return off

        off16 = lax.fori_loop(0, nchunks // 8, p2, zeros)
        len_max = jnp.max(off16)

        def p3(i, _):
            bk = plsc.load_gather(kbuf, [iota + i * _L])
            ub2 = lax.shift_right_logical(bk, 8) & 0xFFF
            plsc.addupdate_scatter(hist, [ub2], ones, mask=i < off16)
            return 0

        lax.fori_loop(0, len_max, p3, 0)

        r1 = k - ca1
        b2, ca2, n2 = _scan4096(hist, r1, iota, zeros)

        def p3z(i, _):
            bk = plsc.load_gather(kbuf, [iota + i * _L])
            ub2 = lax.shift_right_logical(bk, 8) & 0xFFF
            plsc.store_scatter(hist, [ub2], zeros, mask=i < off16)
            return 0

        lax.fori_loop(0, len_max, p3z, 0)

        def p4(i, mb):
            bk = plsc.load_gather(kbuf, [iota + i * _L])
            valid = i < off16
            ub2 = lax.shift_right_logical(bk, 8) & 0xFFF
            mb = jnp.maximum(
                mb, jnp.where(valid & (ub2 < b2), bk ^ _SIGN, _IMIN)
            )
            plsc.addupdate_scatter(
                hist, [bk & 0xFF], ones, mask=valid & (ub2 == b2)
            )
            return mb

        mb2f = lax.fori_loop(0, len_max, p4, zeros + _IMIN)

        r2 = r1 - ca2
        _, lo_k, _ = _scan256(hist, r2, iota, zeros)
        fnd_k1, lo_k1, _ = _scan256(hist, r2 + 1, iota, zeros)

        def p4z(i, _):
            bk = plsc.load_gather(kbuf, [iota + i * _L])
            valid = i < off16
            ub2 = lax.shift_right_logical(bk, 8) & 0xFFF
            plsc.store_scatter(
                hist, [bk & 0xFF], zeros, mask=valid & (ub2 == b2)
            )
            return 0

        lax.fori_loop(0, len_max, p4z, 0)

        prefix = (b1 << 20) | (b2 << 8)
        bk_k = prefix | lo_k
        mbf = jnp.maximum(jnp.max(mbs[pl.ds(0, _L)]), jnp.max(mb2f))
        bk_k1 = jnp.where(fnd_k1 > 0, prefix | lo_k1, mbf ^ _SIGN)

        def unkey(s):
            t = lax.shift_right_arithmetic(s, 31)
            return lax.bitcast_convert_type(s ^ ((t ^ -1) | _SIGN), jnp.float32)

        thr = (unkey(bk_k) + unkey(bk_k1)) * jnp.float32(0.5)

        def p5(i, _):
            for u in range(8):
                sl = pl.ds(i * 128 + u * _L, _L)
                v = xbuf[sl]
                xbuf[sl] = jnp.where(v > thr, jnp.float32(1.0), jnp.float32(0.0))
            return 0

        lax.fori_loop(0, nchunks // 8, p5, 0)

        pltpu.sync_copy(xbuf, o_hbm.at[row])


def _sc_half(x, k):
    m, n = x.shape
    rows_per_worker = m // _NW
    mesh = plsc.VectorSubcoreMesh(
        core_axis_name="c", subcore_axis_name="s", num_cores=_NC, num_subcores=_NS
    )
    body = functools.partial(_sc_body, n=n, k=k, rows_per_worker=rows_per_worker)
    return pl.kernel(
        body,
        out_type=jax.ShapeDtypeStruct((m, n), jnp.float32),
        mesh=mesh,
        compiler_params=pltpu.CompilerParams(needs_layout_passes=False),
        scratch_types=[
            pltpu.VMEM((n,), jnp.float32),
            pltpu.VMEM((n,), jnp.int32),
            pltpu.VMEM((4096,), jnp.int32),
            pltpu.VMEM((_L,), jnp.int32),
        ],
    )(x)


def _order_key_u32(x):
    b = lax.bitcast_convert_type(x, jnp.int32)
    flip = lax.shift_right_arithmetic(b, 31) | jnp.int32(_SIGN)
    return lax.bitcast_convert_type(b ^ flip, jnp.uint32)


def _key_to_f32(u):
    ui = lax.bitcast_convert_type(u, jnp.int32)
    flip = ~lax.shift_right_arithmetic(ui, 31) | jnp.int32(_SIGN)
    return lax.bitcast_convert_type(ui ^ flip, jnp.float32)


def _kwta_block(x_ref, o_ref, *, k):
    x = x_ref[...]
    ukey = _order_key_u32(x)
    m = x.shape[0]
    zero = jnp.zeros((m, 1), jnp.uint32)

    def body(i, carry):
        t1, t2 = carry
        bit = jnp.uint32(31) - jnp.uint32(i)
        add = lax.shift_left(jnp.uint32(1), bit)
        c1 = t1 | add
        c2 = t2 | add
        n1 = jnp.sum((ukey >= c1).astype(jnp.int32), axis=1, keepdims=True)
        n2 = jnp.sum((ukey >= c2).astype(jnp.int32), axis=1, keepdims=True)
        t1 = jnp.where(n1 >= k, c1, t1)
        t2 = jnp.where(n2 >= k + 1, c2, t2)
        return t1, t2

    t1, t2 = lax.fori_loop(0, 32, body, (zero, zero))
    thr = (_key_to_f32(t1) + _key_to_f32(t2)) * jnp.float32(0.5)
    o_ref[...] = (x > thr).astype(jnp.float32)


def _tc_half(x, k):
    m, n = x.shape
    grid = (m // _BLOCK_M,)
    return pl.pallas_call(
        lambda x_ref, o_ref: _kwta_block(x_ref, o_ref, k=k),
        grid=grid,
        in_specs=[pl.BlockSpec((_BLOCK_M, n), lambda i: (i, 0))],
        out_specs=pl.BlockSpec((_BLOCK_M, n), lambda i: (i, 0)),
        out_shape=jax.ShapeDtypeStruct((m, n), jnp.float32),
    )(x)


def kernel(x):
    m, n = x.shape
    k = math.ceil(_SPARSITY * n)
    ms = (m // 2) // _NW * _NW
    if ms == 0:
        return _tc_half(x, k)
    sc_out = _sc_half(x[:ms], k)
    tc_out = _tc_half(x[ms:], k)
    return jnp.concatenate([sc_out, tc_out], axis=0)


# speedup vs baseline: 1.7489x; 1.0642x over previous
import functools
import math

import jax
import jax.numpy as jnp
from jax import lax
from jax.experimental import pallas as pl
from jax.experimental.pallas import tpu as pltpu
from jax.experimental.pallas import tpu_sc as plsc

_SPARSITY = 0.05
_L = 16
_NC = 2
_NS = 16
_NW = _NC * _NS

_IMIN = -2147483648
_SIGN = -2147483648

_BLOCK_M = 8


def _suffix(v):
    return lax.rev(plsc.cumsum(lax.rev(v, (0,))), (0,))


def _cross(g, ga, r):
    return ((g >= r) & (ga < r)).astype(jnp.int32)


def _scan4096(hist, r, iota, zeros):
    seg_stride = iota * 256

    def l0(j, seg):
        return seg + plsc.load_gather(hist, [seg_stride + j])

    seg = lax.fori_loop(0, 256, l0, zeros)
    rc = _suffix(seg)
    ga = rc - seg
    ci = _cross(rc, ga, r)
    s1 = jnp.sum(ci * iota)
    t1 = jnp.sum(ci * ga)

    base1 = s1 * 256

    def l1(m, sub):
        return sub + plsc.load_gather(hist, [base1 + iota * _L + m])

    sub = lax.fori_loop(0, _L, l1, zeros)
    rc2 = t1 + _suffix(sub)
    ga2 = rc2 - sub
    ci2 = _cross(rc2, ga2, r)
    s2 = jnp.sum(ci2 * iota)
    t2 = jnp.sum(ci2 * ga2)

    base2 = base1 + s2 * _L
    h = plsc.load_gather(hist, [base2 + iota])
    rc3 = t2 + _suffix(h)
    ga3 = rc3 - h
    ci3 = _cross(rc3, ga3, r)
    b = jnp.sum(ci3 * (base2 + iota))
    return b, jnp.sum(ci3 * ga3), jnp.sum(ci3 * h)


def _scan256(hist, r, iota, zeros):
    seg_stride = iota * _L

    def l0(j, seg):
        return seg + plsc.load_gather(hist, [seg_stride + j])

    seg = lax.fori_loop(0, _L, l0, zeros)
    rc = _suffix(seg)
    ga = rc - seg
    ci = _cross(rc, ga, r)
    s1 = jnp.sum(ci * iota)
    t1 = jnp.sum(ci * ga)

    h = plsc.load_gather(hist, [s1 * _L + iota])
    rc2 = t1 + _suffix(h)
    ga2 = rc2 - h
    ci2 = _cross(rc2, ga2, r)
    b = jnp.sum(ci2 * (s1 * _L + iota))
    return jnp.sum(ci2), b, jnp.sum(ci2 * ga2)


def _sc_body(x_hbm, o_hbm, xbuf, kbuf, hist, mbs, *, n, k, rows_per_worker):
    wid = lax.axis_index("s") * _NC + lax.axis_index("c")
    iota = lax.iota(jnp.int32, _L)
    zeros = iota * 0
    ones = zeros + 1
    nchunks = n // _L

    def zero_hist(nbkt):
        def z(i, _):
            hist[pl.ds(i * _L, _L)] = zeros
            return 0

        lax.fori_loop(0, nbkt // _L, z, 0)

    zero_hist(4096)

    for rr in range(rows_per_worker):
        row = wid * rows_per_worker + rr
        pltpu.sync_copy(x_hbm.at[row], xbuf)

        def p1(i, _):
            for u in range(8):
                sl = pl.ds(i * 128 + u * _L, _L)
                b = lax.bitcast_convert_type(xbuf[sl], jnp.int32)
                bk = b ^ (lax.shift_right_arithmetic(b, 31) | _SIGN)
                kbuf[sl] = bk
                plsc.addupdate_scatter(
                    hist, [lax.shift_right_logical(bk, 20)], ones
                )
            return 0

        lax.fori_loop(0, nchunks // 8, p1, 0)

        b1, ca1, n1 = _scan4096(hist, jnp.int32(k), iota, zeros)
        zero_hist(4096)

        mbs[pl.ds(0, _L)] = zeros + _IMIN

        @pl.when(ca1 + n1 == k)
        def _():
            def pmb(i, m):
                for u in range(8):
                    bk = kbuf[pl.ds(i * 128 + u * _L, _L)]
                    ub = lax.shift_right_logical(bk, 20)
                    m = jnp.maximum(m, jnp.where(ub < b1, bk ^ _SIGN, _IMIN))
                return m

            mbs[pl.ds(0, _L)] = lax.fori_loop(
                0, nchunks // 8, pmb, zeros + _IMIN
            )

        def p2(i, off):
            for u in range(8):
                bk = kbuf[pl.ds(i * 128 + u * _L, _L)]
                match = lax.shift_right_logical(bk, 20) == b1
                plsc.store_scatter(kbuf, [iota + off * _L], bk, mask=match)
                off = off + match
            return off

        off16 = lax.fori_loop(0, nchunks // 8, p2, zeros)
        len_max = jnp.max(off16)

        def p3(i, _):
            bk = plsc.load_gather(kbuf, [iota + i * _L])
            ub2 = lax.shift_right_logical(bk, 8) & 0xFFF
            plsc.addupdate_scatter(hist, [ub2], ones, mask=i < off16)
            return 0

        lax.fori_loop(0, len_max, p3, 0)

        r1 = k - ca1
        b2, ca2, n2 = _scan4096(hist, r1, iota, zeros)

        def p3z(i, _):
            bk = plsc.load_gather(kbuf, [iota + i * _L])
            ub2 = lax.shift_right_logical(bk, 8) & 0xFFF
            plsc.store_scatter(hist, [ub2], zeros, mask=i < off16)
            return 0

        lax.fori_loop(0, len_max, p3z, 0)

        def p4(i, mb):
            bk = plsc.load_gather(kbuf, [iota + i * _L])
            valid = i < off16
            ub2 = lax.shift_right_logical(bk, 8) & 0xFFF
            mb = jnp.maximum(
                mb, jnp.where(valid & (ub2 < b2), bk ^ _SIGN, _IMIN)
            )
            plsc.addupdate_scatter(
                hist, [bk & 0xFF], ones, mask=valid & (ub2 == b2)
            )
            return mb

        mb2f = lax.fori_loop(0, len_max, p4, zeros + _IMIN)

        r2 = r1 - ca2
        _, lo_k, _ = _scan256(hist, r2, iota, zeros)
        fnd_k1, lo_k1, _ = _scan256(hist, r2 + 1, iota, zeros)

        def p4z(i, _):
            bk = plsc.load_gather(kbuf, [iota + i * _L])
            valid = i < off16
            ub2 = lax.shift_right_logical(bk, 8) & 0xFFF
            plsc.store_scatter(
                hist, [bk & 0xFF], zeros, mask=valid & (ub2 == b2)
            )
            return 0

        lax.fori_loop(0, len_max, p4z, 0)

        prefix = (b1 << 20) | (b2 << 8)
        bk_k = prefix | lo_k
        mbf = jnp.maximum(jnp.max(mbs[pl.ds(0, _L)]), jnp.max(mb2f))
        bk_k1 = jnp.where(fnd_k1 > 0, prefix | lo_k1, mbf ^ _SIGN)

        def unkey(s):
            t = lax.shift_right_arithmetic(s, 31)
            return lax.bitcast_convert_type(s ^ ((t ^ -1) | _SIGN), jnp.float32)

        thr = (unkey(bk_k) + unkey(bk_k1)) * jnp.float32(0.5)

        def p5(i, _):
            for u in range(8):
                sl = pl.ds(i * 128 + u * _L, _L)
                v = xbuf[sl]
                xbuf[sl] = jnp.where(v > thr, jnp.float32(1.0), jnp.float32(0.0))
            return 0

        lax.fori_loop(0, nchunks // 8, p5, 0)

        pltpu.sync_copy(xbuf, o_hbm.at[row])


def _sc_half(x, k, ms):
    _, n = x.shape
    rows_per_worker = ms // _NW
    mesh = plsc.VectorSubcoreMesh(
        core_axis_name="c", subcore_axis_name="s", num_cores=_NC, num_subcores=_NS
    )
    body = functools.partial(_sc_body, n=n, k=k, rows_per_worker=rows_per_worker)
    return pl.kernel(
        body,
        out_type=jax.ShapeDtypeStruct((ms, n), jnp.float32),
        mesh=mesh,
        compiler_params=pltpu.CompilerParams(needs_layout_passes=False),
        scratch_types=[
            pltpu.VMEM((n,), jnp.float32),
            pltpu.VMEM((n,), jnp.int32),
            pltpu.VMEM((4096,), jnp.int32),
            pltpu.VMEM((_L,), jnp.int32),
        ],
    )(x)


def _order_key_u32(x):
    b = lax.bitcast_convert_type(x, jnp.int32)
    flip = lax.shift_right_arithmetic(b, 31) | jnp.int32(_SIGN)
    return lax.bitcast_convert_type(b ^ flip, jnp.uint32)


def _key_to_f32(u):
    ui = lax.bitcast_convert_type(u, jnp.int32)
    flip = ~lax.shift_right_arithmetic(ui, 31) | jnp.int32(_SIGN)
    return lax.bitcast_convert_type(ui ^ flip, jnp.float32)


def _kwta_block(x_ref, o_ref, *, k):
    x = x_ref[...]
    ukey = _order_key_u32(x)
    m = x.shape[0]
    zero = jnp.zeros((m, 1), jnp.uint32)

    def body(i, carry):
        t1, t2 = carry
        bit = jnp.uint32(31) - jnp.uint32(i)
        add = lax.shift_left(jnp.uint32(1), bit)
        c1 = t1 | add
        c2 = t2 | add
        n1 = jnp.sum((ukey >= c1).astype(jnp.int32), axis=1, keepdims=True)
        n2 = jnp.sum((ukey >= c2).astype(jnp.int32), axis=1, keepdims=True)
        t1 = jnp.where(n1 >= k, c1, t1)
        t2 = jnp.where(n2 >= k + 1, c2, t2)
        return t1, t2

    t1, t2 = lax.fori_loop(0, 32, body, (zero, zero))
    thr = (_key_to_f32(t1) + _key_to_f32(t2)) * jnp.float32(0.5)
    o_ref[...] = (x > thr).astype(jnp.float32)


def _tc_half(x, k, ms):
    m, n = x.shape
    off = ms // _BLOCK_M
    grid = ((m - ms) // _BLOCK_M,)
    return pl.pallas_call(
        lambda x_ref, o_ref: _kwta_block(x_ref, o_ref, k=k),
        grid=grid,
        in_specs=[pl.BlockSpec((_BLOCK_M, n), lambda i: (i + off, 0))],
        out_specs=pl.BlockSpec((_BLOCK_M, n), lambda i: (i, 0)),
        out_shape=jax.ShapeDtypeStruct((m - ms, n), jnp.float32),
    )(x)


def kernel(x):
    m, n = x.shape
    k = math.ceil(_SPARSITY * n)
    ms = (m // 2) // _NW * _NW
    if ms == 0:
        return _tc_half(x, k, 0)
    sc_out = _sc_half(x, k, ms)
    tc_out = _tc_half(x, k, ms)
    return jnp.concatenate([sc_out, tc_out], axis=0)
